# Initial kernel scaffold; baseline (speedup 1.0000x reference)
import jax
import jax.numpy as jnp
from jax.experimental import pallas as pl


def kernel(output_pc, gt_pc):
    raise NotImplementedError("write your pallas kernel here")


# speedup vs baseline: 1.0415x; 1.0415x over previous
import functools

import jax
import jax.numpy as jnp
from jax import lax
from jax.experimental import pallas as pl
from jax.experimental.pallas import tpu as pltpu
from jax.experimental.pallas import tpu_sc as plsc

N = 8192
NCHUNK = 4
CHUNK = N // NCHUNK
NW = 32
PER_W = N // NW
L = 16
G = 8


def _derive(raw_x, raw_y, raw_z, d_n, d_xm2, d_ym2, d_zm2):
    def body(v, _):
        sl = pl.ds(v * L, L)
        x = raw_x[sl]
        y = raw_y[sl]
        z = raw_z[sl]
        d_n[sl] = x * x + y * y + z * z
        d_xm2[sl] = -2.0 * x
        d_ym2[sl] = -2.0 * y
        d_zm2[sl] = -2.0 * z
        return 0

    lax.fori_loop(0, N // L, body, 0)


def _direction(wid, s, own_x, own_y, own_z, own_n,
               sc_n, sc_xm2, sc_ym2, sc_zm2):
    def per_block(jb, s):
        jbase = wid * PER_W + jb * L
        vjx = own_x[pl.ds(jbase, L)]
        vjy = own_y[pl.ds(jbase, L)]
        vjz = own_z[pl.ds(jbase, L)]
        vjn = own_n[pl.ds(jbase, L)]
        for h in range(L // G):
            bx = [vjx[h * G + g] for g in range(G)]
            by = [vjy[h * G + g] for g in range(G)]
            bz = [vjz[h * G + g] for g in range(G)]
            for c in range(NCHUNK):

                def body(iv, maccs):
                    sl = pl.ds(iv * L, L)
                    vn = sc_n[sl]
                    vx = sc_xm2[sl]
                    vy = sc_ym2[sl]
                    vz = sc_zm2[sl]
                    out = []
                    for g in range(G):
                        t = vn + vx * bx[g] + vy * by[g] + vz * bz[g]
                        out.append(jnp.minimum(maccs[g], t))
                    return tuple(out)

                inf = jnp.full((L,), jnp.inf, dtype=jnp.float32)
                maccs = lax.fori_loop(c * (CHUNK // L), (c + 1) * (CHUNK // L),
                                      body, (inf,) * G)
                for g in range(G):
                    s = s + jnp.min(maccs[g]) + vjn[h * G + g]
        return s

    return lax.fori_loop(0, PER_W // L, per_block, s)


def _chamfer_body(x1_hbm, y1_hbm, z1_hbm, x2_hbm, y2_hbm, z2_hbm, out_hbm,
                  c1x, c1y, c1z, c1n, c1xm2, c1ym2, c1zm2,
                  c2x, c2y, c2z, c2n, c2xm2, c2ym2, c2zm2,
                  svec):
    wid = lax.axis_index("s") * 2 + lax.axis_index("c")

    pltpu.sync_copy(x1_hbm, c1x)
    pltpu.sync_copy(y1_hbm, c1y)
    pltpu.sync_copy(z1_hbm, c1z)
    pltpu.sync_copy(x2_hbm, c2x)
    pltpu.sync_copy(y2_hbm, c2y)
    pltpu.sync_copy(z2_hbm, c2z)

    _derive(c1x, c1y, c1z, c1n, c1xm2, c1ym2, c1zm2)
    _derive(c2x, c2y, c2z, c2n, c2xm2, c2ym2, c2zm2)

    s = jnp.float32(0.0)
    s = _direction(wid, s, c2x, c2y, c2z, c2n, c1n, c1xm2, c1ym2, c1zm2)
    s = _direction(wid, s, c1x, c1y, c1z, c1n, c2n, c2xm2, c2ym2, c2zm2)

    svec[...] = jnp.full((L,), s * (1.0 / L), dtype=jnp.float32)
    pltpu.sync_copy(svec, out_hbm.at[wid])


@jax.jit
def _chamfer_sc(x1, y1, z1, x2, y2, z2):
    mesh = plsc.VectorSubcoreMesh(core_axis_name="c", subcore_axis_name="s")
    vec = pltpu.VMEM((N,), jnp.float32)
    run = pl.kernel(
        _chamfer_body,
        out_type=jax.ShapeDtypeStruct((NW, L), jnp.float32),
        mesh=mesh,
        scratch_types=[vec] * 14 + [pltpu.VMEM((L,), jnp.float32)],
        compiler_params=pltpu.CompilerParams(needs_layout_passes=False),
    )
    return run(x1, y1, z1, x2, y2, z2)


def kernel(output_pc, gt_pc):
    p1 = jnp.squeeze(output_pc)
    p2 = jnp.squeeze(gt_pc)
    partials = _chamfer_sc(p1[:, 0], p1[:, 1], p1[:, 2],
                           p2[:, 0], p2[:, 1], p2[:, 2])
    return jnp.sum(partials) / (NCHUNK * N)


# speedup vs baseline: 1.5470x; 1.4853x over previous
import functools

import jax
import jax.numpy as jnp
from jax import lax
from jax.experimental import pallas as pl
from jax.experimental.pallas import tpu as pltpu
from jax.experimental.pallas import tpu_sc as plsc

N = 8192
NCHUNK = 4
CHUNK = N // NCHUNK
NW = 32
NS = 16
PER_W = N // NW
L = 16
G = 8
WPC = CHUNK // PER_W
IRED = N // NS


def _norms(px, py, pz, d_n):
    def body(v, _):
        sl = pl.ds(v * L, L)
        x = px[sl]
        y = py[sl]
        z = pz[sl]
        d_n[sl] = x * x + y * y + z * z
        return 0

    lax.fori_loop(0, N // L, body, 0)


def _derive(px, py, pz, d_n, d_xm2, d_ym2, d_zm2):
    def body(v, _):
        sl = pl.ds(v * L, L)
        x = px[sl]
        y = py[sl]
        z = pz[sl]
        d_n[sl] = x * x + y * y + z * z
        d_xm2[sl] = -2.0 * x
        d_ym2[sl] = -2.0 * y
        d_zm2[sl] = -2.0 * z
        return 0

    lax.fori_loop(0, N // L, body, 0)


def _chamfer_body(x1_hbm, y1_hbm, z1_hbm, x2_hbm, y2_hbm, z2_hbm, out_hbm,
                  c1x, c1y, c1z, c1n, c1xm2, c1ym2, c1zm2,
                  c2x, c2y, c2z, c2n,
                  m2part, comb, svec, shared):
    sid = lax.axis_index("s")
    cid = lax.axis_index("c")
    wid = cid * NS + sid

    pltpu.sync_copy(x1_hbm, c1x)
    pltpu.sync_copy(y1_hbm, c1y)
    pltpu.sync_copy(z1_hbm, c1z)
    pltpu.sync_copy(x2_hbm, c2x)
    pltpu.sync_copy(y2_hbm, c2y)
    pltpu.sync_copy(z2_hbm, c2z)

    _derive(c1x, c1y, c1z, c1n, c1xm2, c1ym2, c1zm2)
    _norms(c2x, c2y, c2z, c2n)

    inf = jnp.full((L,), jnp.inf, dtype=jnp.float32)

    def initm2(v, _):
        m2part[pl.ds(v * L, L)] = inf
        return 0

    lax.fori_loop(0, N // L, initm2, 0)

    def per_block(jb, s):
        jbase = wid * PER_W + jb * L
        vjx = c2x[pl.ds(jbase, L)]
        vjy = c2y[pl.ds(jbase, L)]
        vjz = c2z[pl.ds(jbase, L)]
        vjn = c2n[pl.ds(jbase, L)]
        for h in range(L // G):
            bx = [vjx[h * G + g] for g in range(G)]
            by = [vjy[h * G + g] for g in range(G)]
            bz = [vjz[h * G + g] for g in range(G)]
            bn = [vjn[h * G + g] for g in range(G)]
            for c in range(NCHUNK):

                def body(iv, maccs):
                    sl = pl.ds(iv * L, L)
                    vn = c1n[sl]
                    vx = c1xm2[sl]
                    vy = c1ym2[sl]
                    vz = c1zm2[sl]
                    acc2 = m2part[sl]
                    out = []
                    for g in range(G):
                        w = vx * bx[g] + vy * by[g] + vz * bz[g]
                        out.append(jnp.minimum(maccs[g], vn + w))
                        acc2 = jnp.minimum(acc2, w + bn[g])
                    m2part[sl] = acc2
                    return tuple(out)

                maccs = lax.fori_loop(c * (CHUNK // L), (c + 1) * (CHUNK // L),
                                      body, (inf,) * G)
                for g in range(G):
                    s = s + jnp.min(maccs[g]) + bn[g]
        return s

    s = lax.fori_loop(0, PER_W // L, per_block, jnp.float32(0.0))

    pltpu.sync_copy(m2part, shared.at[sid])
    plsc.subcore_barrier()
    for cc in range(NS // WPC):
        pltpu.sync_copy(
            shared.at[pl.ds(cc * WPC, WPC), pl.ds(sid * IRED, IRED)], comb)

        def red(v, s):
            sl = pl.ds(v * L, L)
            m = comb[0, sl]
            for r in range(1, WPC):
                m = jnp.minimum(m, comb[r, sl])
            m = m + c1n[pl.ds(sid * IRED + v * L, L)]
            return s + jnp.sum(m)

        s = lax.fori_loop(0, IRED // L, red, s)

    svec[...] = jnp.full((L,), s * (1.0 / L), dtype=jnp.float32)
    pltpu.sync_copy(svec, out_hbm.at[wid])


@jax.jit
def _chamfer_sc(x1, y1, z1, x2, y2, z2):
    mesh = plsc.VectorSubcoreMesh(core_axis_name="c", subcore_axis_name="s")
    vec = pltpu.VMEM((N,), jnp.float32)
    run = pl.kernel(
        _chamfer_body,
        out_type=jax.ShapeDtypeStruct((NW, L), jnp.float32),
        mesh=mesh,
        scratch_types=[vec] * 11 + [
            pltpu.VMEM((N,), jnp.float32),
            pltpu.VMEM((WPC, IRED), jnp.float32),
            pltpu.VMEM((L,), jnp.float32),
            pltpu.MemorySpace.VMEM_SHARED((NS, N), jnp.float32),
        ],
        compiler_params=pltpu.CompilerParams(needs_layout_passes=False),
    )
    return run(x1, y1, z1, x2, y2, z2)


def kernel(output_pc, gt_pc):
    p1 = jnp.squeeze(output_pc)
    p2 = jnp.squeeze(gt_pc)
    partials = _chamfer_sc(p1[:, 0], p1[:, 1], p1[:, 2],
                           p2[:, 0], p2[:, 1], p2[:, 2])
    return jnp.sum(partials) / (NCHUNK * N)
